# baseline (speedup 1.0000x reference)
import jax
import jax.numpy as jnp
from jax.experimental import pallas as pl
from jax.experimental.pallas import tpu as pltpu
from jax.experimental.pallas import tpu_sc as plsc


_BLK_S = 1024


def _add_kernel(x_ref, pos_ref, o_ref):
    o_ref[...] = x_ref[...] + pos_ref[...]


def _tc_kernel(x, pos_table):
    batch, seq_len, dim = x.shape
    blk = _BLK_S
    grid = (seq_len // blk, batch)
    return pl.pallas_call(
        _add_kernel,
        grid=grid,
        in_specs=[
            pl.BlockSpec((1, blk, dim), lambda s, b: (b, s, 0)),
            pl.BlockSpec((blk, dim), lambda s, b: (s, 0)),
        ],
        out_specs=pl.BlockSpec((1, blk, dim), lambda s, b: (b, s, 0)),
        out_shape=jax.ShapeDtypeStruct((batch, seq_len, dim), x.dtype),
    )(x, pos_table)


_SC_LANES = 16
_SC_BLK_R = 8


def _sc_block_body(x_vmem, pos_vmem, o_vmem):
    @pl.loop(0, _SC_BLK_R)
    def _row(r):
        @pl.loop(0, 2048, step=_SC_LANES)
        def _col(c):
            slc = (pl.ds(r, 1), pl.ds(c, _SC_LANES))
            o_vmem.at[slc][...] = x_vmem.at[slc][...] + pos_vmem.at[slc][...]


def _sc_kernel(x, pos_table):
    batch, seq_len, dim = x.shape
    x2d = x.reshape(batch * seq_len, dim)
    rows = batch * seq_len
    blk_r = _SC_BLK_R
    seq_blocks = seq_len // blk_r
    mesh = plsc.VectorSubcoreMesh(core_axis_name="core", subcore_axis_name="subcore")

    @pl.kernel(out_type=jax.ShapeDtypeStruct((rows, dim), x.dtype), mesh=mesh)
    def run(x_hbm, pos_hbm, o_hbm):
        pltpu.emit_pipeline(
            _sc_block_body,
            grid=(rows // blk_r,),
            in_specs=[
                pl.BlockSpec((blk_r, dim), index_map=lambda i: (i, 0)),
                pl.BlockSpec((blk_r, dim), index_map=lambda i: (i % seq_blocks, 0)),
            ],
            out_specs=[pl.BlockSpec((blk_r, dim), index_map=lambda i: (i, 0))],
            core_axis_name=("core", "subcore"),
            dimension_semantics=(pltpu.PARALLEL,),
        )(x_hbm, pos_hbm, o_hbm)

    return run(x2d, pos_table).reshape(batch, seq_len, dim)


def kernel(x, pos_table):
    return _sc_kernel(x, pos_table)


# speedup vs baseline: 1.0052x; 1.0052x over previous
import jax
import jax.numpy as jnp
from jax.experimental import pallas as pl
from jax.experimental.pallas import tpu as pltpu
from jax.experimental.pallas import tpu_sc as plsc


_BLK_S = 1024


def _add_kernel(x_ref, pos_ref, o_ref):
    o_ref[...] = x_ref[...] + pos_ref[...]


def _tc_kernel(x, pos_table):
    batch, seq_len, dim = x.shape
    blk = _BLK_S
    grid = (seq_len // blk, batch)
    return pl.pallas_call(
        _add_kernel,
        grid=grid,
        in_specs=[
            pl.BlockSpec((1, blk, dim), lambda s, b: (b, s, 0)),
            pl.BlockSpec((blk, dim), lambda s, b: (s, 0)),
        ],
        out_specs=pl.BlockSpec((1, blk, dim), lambda s, b: (b, s, 0)),
        out_shape=jax.ShapeDtypeStruct((batch, seq_len, dim), x.dtype),
    )(x, pos_table)


_SC_LANES = 16
_SC_BLK_R = 8


def _sc_block_body(x_vmem, pos_vmem, idx_vmem, o_vmem):
    @pl.loop(0, _SC_BLK_R)
    def _row(r):
        for c in range(0, 2048, _SC_LANES):
            slc = (pl.ds(r, 1), pl.ds(c, _SC_LANES))
            o_vmem.at[slc][...] = x_vmem.at[slc][...] + pos_vmem.at[slc][...]


def _sc_kernel(x, pos_table):
    batch, seq_len, dim = x.shape
    x2d = x.reshape(batch * seq_len, dim)
    rows = batch * seq_len
    blk_r = _SC_BLK_R
    seq_blocks = seq_len // blk_r
    idx = jnp.arange(blk_r, dtype=jnp.int32).reshape(1, blk_r)
    mesh = plsc.VectorSubcoreMesh(core_axis_name="core", subcore_axis_name="subcore")

    @pl.kernel(out_type=jax.ShapeDtypeStruct((rows, dim), x.dtype), mesh=mesh)
    def run(x_hbm, pos_hbm, idx_hbm, o_hbm):
        pltpu.emit_pipeline(
            _sc_block_body,
            grid=(rows // blk_r,),
            in_specs=[
                pl.BlockSpec((blk_r, dim), index_map=lambda i: (i, 0)),
                pl.BlockSpec((blk_r, dim), index_map=lambda i: (i % seq_blocks, 0)),
                pl.BlockSpec((1, blk_r), index_map=lambda i: (0, 0)),
            ],
            out_specs=[pl.BlockSpec((blk_r, dim), index_map=lambda i: (i, 0))],
            core_axis_name=("core", "subcore"),
            dimension_semantics=(pltpu.PARALLEL,),
        )(x_hbm, pos_hbm, idx_hbm, o_hbm)

    return run(x2d, pos_table, idx).reshape(batch, seq_len, dim)


def kernel(x, pos_table):
    return _sc_kernel(x, pos_table)


# speedup vs baseline: 3.9501x; 3.9298x over previous
import jax
import jax.numpy as jnp
from jax.experimental import pallas as pl


_BLK_S = 1024


def _add_kernel(x_ref, pos_ref, o_ref):
    o_ref[...] = x_ref[...] + pos_ref[...]


def kernel(x, pos_table):
    batch, seq_len, dim = x.shape
    blk = _BLK_S
    grid = (seq_len // blk, batch)
    return pl.pallas_call(
        _add_kernel,
        grid=grid,
        in_specs=[
            pl.BlockSpec((1, blk, dim), lambda s, b: (b, s, 0)),
            pl.BlockSpec((blk, dim), lambda s, b: (s, 0)),
        ],
        out_specs=pl.BlockSpec((1, blk, dim), lambda s, b: (b, s, 0)),
        out_shape=jax.ShapeDtypeStruct((batch, seq_len, dim), x.dtype),
    )(x, pos_table)
